# baseline (speedup 1.0000x reference)
import dataclasses

import jax
import jax.numpy as jnp
from jax import lax
from jax.experimental import pallas as pl
from jax.experimental.pallas import tpu as pltpu
from jax.experimental.pallas import tpu_sc as plsc

_B = 512
_D = 128
_NC = 2
_NS = 16
_L = 16
_BPW = _B // (_NC * _NS)
_F1 = 1.0
_FM = 0.1
_UNROLL = 8


def _rsqrt(x):
    i = lax.bitcast_convert_type(x, jnp.int32)
    y = lax.bitcast_convert_type(jnp.int32(0x5F3759DF) - (i >> 1), jnp.float32)
    for _ in range(3):
        y = y * (jnp.float32(1.5) - jnp.float32(0.5) * x * y * y)
    return y


def _sqrt(x):
    return x * _rsqrt(jnp.maximum(x, jnp.float32(1e-30)))


def _relu(x):
    return jnp.maximum(x, jnp.float32(0.0))


def _reg(n2):
    return jnp.abs(_sqrt(n2) - _F1)


_CSLOTS = [(0, 0), (0, 1), (1, 0), (1, 1), (1, 2), (2, 0), (2, 2),
           (3, 1), (3, 2), (4, 0), (4, 1), (7, 0), (7, 2)]
_RSLOTS = [(2, 1), (3, 0), (5, 0), (5, 1), (6, 0), (6, 1), (6, 2), (7, 1)]


def _body(nf1_h, nf2_h, nf3_h, nf4_h, dis_h, ri_h, rch_h, neg_h, cx_h,
          cer_h, re_h, out_h, ib0, ib1, ib2, ib3, ib4, ib5, ib6, ib7,
          cblk, rblk, crows, rrows, radii, accbuf, tmp, outv, shared,
          sem_i, sem_r, sem_a, sem_b, sem_c):
    ib = (ib0, ib1, ib2, ib3, ib4, ib5, ib6, ib7)
    cid = lax.axis_index("c")
    sid = lax.axis_index("s")
    wid = cid * _NS + sid
    base = wid * _BPW

    iota = lax.iota(jnp.int32, _L)

    h_ib = [pltpu.async_copy(src.at[pl.ds(base, _BPW)], dst, sem_i)
            for src, dst in zip((nf1_h, nf2_h, nf3_h, nf4_h, dis_h, ri_h,
                                 rch_h, neg_h), ib)]
    h_rad = pltpu.async_copy(cer_h, radii, sem_r)
    for h in h_ib:
        h.wait()

    def _xcol(k, col):
        return plsc.load_gather(ib[k], [iota, jnp.full((_L,), col, jnp.int32)])

    cidx = [_xcol(k, col) for k, col in _CSLOTS]
    ridx = [_xcol(k, col) for k, col in _RSLOTS]
    for t, v in enumerate(cidx):
        r, c = divmod(t, 7)
        cblk[r, pl.ds(c * _L, _L)] = v
    cblk[1, pl.ds(6 * _L, _L)] = jnp.zeros((_L,), jnp.int32)
    for t, v in enumerate(ridx):
        rblk[0, pl.ds(t * _L, _L)] = v

    h_a = pltpu.async_copy(cx_h.at[cblk.at[0]], crows.at[pl.ds(0, 112)],
                           sem_a)
    h_b = pltpu.async_copy(cx_h.at[cblk.at[1]], crows.at[pl.ds(112, 112)],
                           sem_b)
    h_c = pltpu.async_copy(re_h.at[rblk.at[0]], rrows, sem_c)

    def _dvec(d):
        return (d + iota) & jnp.int32(127)

    def _ccol(t, dv):
        return plsc.load_gather(crows, [t * _L + iota, dv])

    def _rcol(t, dv):
        return plsc.load_gather(rrows, [t * _L + iota, dv])

    def _rad(t):
        return jnp.abs(plsc.load_gather(radii, [cidx[t]]))

    z = jnp.zeros((_L,), jnp.float32)

    h_a.wait()
    h_rad.wait()

    def l1(d, c):
        s, na, nb = c
        dv = _dvec(d)
        a = _ccol(0, dv); b = _ccol(1, dv)
        t = a - b
        return (s + t * t, na + a * a, nb + b * b)
    s, n1, n2 = lax.fori_loop(0, _D, l1, (z, z, z), unroll=_UNROLL)
    acc = _relu(_sqrt(s) + _rad(0) - _rad(1) - _FM) + _reg(n1) + _reg(n2)

    def l2(d, c):
        d21, d31, d32, n1, n2, n3 = c
        dv = _dvec(d)
        a = _ccol(2, dv); b = _ccol(3, dv); e = _ccol(4, dv)
        t21 = b - a; t31 = e - a; t32 = e - b
        return (d21 + t21 * t21, d31 + t31 * t31, d32 + t32 * t32,
                n1 + a * a, n2 + b * b, n3 + e * e)
    d21, d31, d32, n1, n2, n3 = lax.fori_loop(
        0, _D, l2, (z, z, z, z, z, z), unroll=_UNROLL)
    rc = _rad(2); rd = _rad(3)
    acc += (_relu(_sqrt(d21) - (rc + rd) - _FM) + _relu(_sqrt(d31) - rc - _FM)
            + _relu(_sqrt(d32) - rd - _FM) + _reg(n1) + _reg(n2) + _reg(n3))

    h_c.wait()

    def l3(d, c):
        s, na, nb = c
        dv = _dvec(d)
        a = _ccol(5, dv); r = _rcol(0, dv); b = _ccol(6, dv)
        t = a + r - b
        return (s + t * t, na + a * a, nb + b * b)
    s, n1, n2 = lax.fori_loop(0, _D, l3, (z, z, z), unroll=_UNROLL)
    acc += _relu(_sqrt(s) + _rad(5) - _rad(6) - _FM) + _reg(n1) + _reg(n2)

    def l6(d, c):
        s, n1, n2, dt = c
        dv = _dvec(d)
        a = _rcol(2, dv); b = _rcol(3, dv)
        t = b - a
        return (s + t * t, n1 + a * a, n2 + b * b, dt + a * b)
    s, n1, n2, dt = lax.fori_loop(0, _D, l6, (z, z, z, z), unroll=_UNROLL)
    direction = dt / (jnp.maximum(_sqrt(n1), jnp.float32(1e-12))
                      * jnp.maximum(_sqrt(n2), jnp.float32(1e-12)))
    acc += (_relu(_sqrt(s) - _FM) + _reg(n1) + _reg(n2)
            + jnp.abs(_F1 - direction))

    def l7(d, c):
        s, n1, n2, n3, ncd, dt = c
        dv = _dvec(d)
        a = _rcol(4, dv); b = _rcol(5, dv); e = _rcol(6, dv)
        t = e - a - b
        cd = a + b
        return (s + t * t, n1 + a * a, n2 + b * b, n3 + e * e,
                ncd + cd * cd, dt + cd * e)
    s, n1, n2, n3, ncd, dt = lax.fori_loop(
        0, _D, l7, (z, z, z, z, z, z), unroll=_UNROLL)
    direction = dt / (jnp.maximum(_sqrt(ncd), jnp.float32(1e-12))
                      * jnp.maximum(_sqrt(n3), jnp.float32(1e-12)))
    acc += (_relu(_sqrt(s) - _FM) + _reg(n1) + _reg(n2) + _reg(n3)
            + jnp.abs(_F1 - direction))

    h_b.wait()

    def l4(d, c):
        s, na, nb = c
        dv = _dvec(d)
        a = _ccol(7, dv); r = _rcol(1, dv); b = _ccol(8, dv)
        t = a - r - b
        return (s + t * t, na + a * a, nb + b * b)
    s, n1, n2 = lax.fori_loop(0, _D, l4, (z, z, z), unroll=_UNROLL)
    acc += _relu(_sqrt(s) - (_rad(7) + _rad(8)) - _FM) + _reg(n1) + _reg(n2)

    def l5(d, c):
        s, na, nb = c
        dv = _dvec(d)
        a = _ccol(9, dv); b = _ccol(10, dv)
        t = b - a
        return (s + t * t, na + a * a, nb + b * b)
    s, n1, n2 = lax.fori_loop(0, _D, l5, (z, z, z), unroll=_UNROLL)
    acc += _relu((_rad(9) + _rad(10)) - _sqrt(s) + _FM) + _reg(n1) + _reg(n2)

    def l8(d, c):
        s, na, nb = c
        dv = _dvec(d)
        a = _ccol(11, dv); r = _rcol(7, dv); b = _ccol(12, dv)
        t = a + r - b
        return (s + t * t, na + a * a, nb + b * b)
    s, n1, n2 = lax.fori_loop(0, _D, l8, (z, z, z), unroll=_UNROLL)
    acc += (-(_sqrt(s) - _rad(11) - _rad(12)) + _FM) + _reg(n1) + _reg(n2)

    acc = acc * jnp.float32(1.0 / _B)

    accbuf[...] = acc
    pltpu.sync_copy(accbuf, shared.at[pl.ds(sid * _L, _L)])
    plsc.subcore_barrier()

    @pl.when(sid == 0)
    def _():
        pltpu.sync_copy(shared, tmp)
        tot = tmp[pl.ds(0, _L)]
        for s_ in range(1, _NS):
            tot = tot + tmp[pl.ds(s_ * _L, _L)]
        outv[...] = jnp.broadcast_to(jnp.sum(tot), (_L,))
        pltpu.sync_copy(outv, out_h.at[cid])


@jax.jit
def _sc_loss(nf1, nf2, nf3, nf4, disjoint, role_inclusion, role_chain,
             nf3_neg, cx, cer, re):
    cp = pltpu.CompilerParams()
    if "needs_layout_passes" in pltpu.CompilerParams.__dataclass_fields__:
        cp = dataclasses.replace(cp, needs_layout_passes=False)
    run = pl.kernel(
        _body,
        out_type=jax.ShapeDtypeStruct((_NC, _L), jnp.float32),
        mesh=plsc.VectorSubcoreMesh(core_axis_name="c", subcore_axis_name="s"),
        scratch_types=[
            pltpu.VMEM((_BPW, 2), jnp.int32),
            pltpu.VMEM((_BPW, 3), jnp.int32),
            pltpu.VMEM((_BPW, 3), jnp.int32),
            pltpu.VMEM((_BPW, 3), jnp.int32),
            pltpu.VMEM((_BPW, 2), jnp.int32),
            pltpu.VMEM((_BPW, 2), jnp.int32),
            pltpu.VMEM((_BPW, 3), jnp.int32),
            pltpu.VMEM((_BPW, 3), jnp.int32),
            pltpu.VMEM((2, 112), jnp.int32),
            pltpu.VMEM((1, 128), jnp.int32),
            pltpu.VMEM((224, _D), jnp.float32),
            pltpu.VMEM((128, _D), jnp.float32),
            pltpu.VMEM((1000,), jnp.float32),
            pltpu.VMEM((_L,), jnp.float32),
            pltpu.VMEM((_NS * _L,), jnp.float32),
            pltpu.VMEM((_L,), jnp.float32),
            pltpu.VMEM_SHARED((_NS * _L,), jnp.float32),
            pltpu.SemaphoreType.DMA,
            pltpu.SemaphoreType.DMA,
            pltpu.SemaphoreType.DMA,
            pltpu.SemaphoreType.DMA,
            pltpu.SemaphoreType.DMA,
        ],
        compiler_params=cp,
    )
    return run(nf1, nf2, nf3, nf4, disjoint, role_inclusion, role_chain,
               nf3_neg, cx, cer, re)


def kernel(nf1, nf2, nf3, nf4, disjoint, role_inclusion, role_chain,
           nf3_neg, class_emb, rel_emb):
    i32 = jnp.int32
    out = _sc_loss(nf1.astype(i32), nf2.astype(i32), nf3.astype(i32),
                   nf4.astype(i32), disjoint.astype(i32),
                   role_inclusion.astype(i32), role_chain.astype(i32),
                   nf3_neg.astype(i32), class_emb[:, :_D], class_emb[:, _D],
                   rel_emb)
    return out[0, 0] + out[1, 0]


# speedup vs baseline: 1.6819x; 1.6819x over previous
import dataclasses

import jax
import jax.numpy as jnp
from jax import lax
from jax.experimental import pallas as pl
from jax.experimental.pallas import tpu as pltpu
from jax.experimental.pallas import tpu_sc as plsc

_B = 512
_D = 128
_NC = 2
_NS = 16
_L = 16
_BPW = _B // (_NC * _NS)
_NT = 21
_GW = 112
_F1 = 1.0
_FM = 0.1
_UNROLL = 4


def _rsqrt(x):
    i = lax.bitcast_convert_type(x, jnp.int32)
    y = lax.bitcast_convert_type(jnp.int32(0x5F3759DF) - (i >> 1), jnp.float32)
    for _ in range(3):
        y = y * (jnp.float32(1.5) - jnp.float32(0.5) * x * y * y)
    return y


def _sqrt(x):
    return x * _rsqrt(jnp.maximum(x, jnp.float32(1e-30)))


def _relu(x):
    return jnp.maximum(x, jnp.float32(0.0))


def _reg(n2):
    return jnp.abs(_sqrt(n2) - _F1)


def _body(cidx_h, tab_h, cer_h, out_h, blk, rows, radii, accbuf, tmp, outv,
          shared, sem_i, sem_r, sem_a, sem_b, sem_c):
    cid = lax.axis_index("c")
    sid = lax.axis_index("s")
    wid = cid * _NS + sid

    iota = lax.iota(jnp.int32, _L)

    h_idx = pltpu.async_copy(cidx_h.at[wid], blk, sem_i)
    h_rad = pltpu.async_copy(cer_h, radii, sem_r)
    h_idx.wait()
    h_a = pltpu.async_copy(tab_h.at[blk.at[0]], rows.at[pl.ds(0, _GW)], sem_a)
    h_b = pltpu.async_copy(tab_h.at[blk.at[1]], rows.at[pl.ds(_GW, _GW)],
                           sem_b)
    h_c = pltpu.async_copy(tab_h.at[blk.at[2]], rows.at[pl.ds(2 * _GW, _GW)],
                           sem_c)

    def _dvec(d):
        return (d + iota) & jnp.int32(127)

    def _col(t, dv):
        return plsc.load_gather(rows, [t * _L + iota, dv])

    def _rad(t):
        ci = plsc.load_gather(
            blk, [jnp.full((_L,), t * _L // _GW, jnp.int32),
                  t * _L % _GW + iota])
        return jnp.abs(plsc.load_gather(radii, [ci]))

    z = jnp.zeros((_L,), jnp.float32)

    h_a.wait()
    h_rad.wait()

    def l1(d, c):
        s, na, nb = c
        dv = _dvec(d)
        a = _col(0, dv); b = _col(1, dv)
        t = a - b
        return (s + t * t, na + a * a, nb + b * b)
    s, n1, n2 = lax.fori_loop(0, _D, l1, (z, z, z), unroll=_UNROLL)
    acc = _relu(_sqrt(s) + _rad(0) - _rad(1) - _FM) + _reg(n1) + _reg(n2)

    def l2(d, c):
        d21, d31, d32, n1, n2, n3 = c
        dv = _dvec(d)
        a = _col(2, dv); b = _col(3, dv); e = _col(4, dv)
        t21 = b - a; t31 = e - a; t32 = e - b
        return (d21 + t21 * t21, d31 + t31 * t31, d32 + t32 * t32,
                n1 + a * a, n2 + b * b, n3 + e * e)
    d21, d31, d32, n1, n2, n3 = lax.fori_loop(
        0, _D, l2, (z, z, z, z, z, z), unroll=_UNROLL)
    rc = _rad(2); rd = _rad(3)
    acc += (_relu(_sqrt(d21) - (rc + rd) - _FM) + _relu(_sqrt(d31) - rc - _FM)
            + _relu(_sqrt(d32) - rd - _FM) + _reg(n1) + _reg(n2) + _reg(n3))

    h_b.wait()

    def l3(d, c):
        s, na, nb = c
        dv = _dvec(d)
        a = _col(5, dv); r = _col(6, dv); b = _col(7, dv)
        t = a + r - b
        return (s + t * t, na + a * a, nb + b * b)
    s, n1, n2 = lax.fori_loop(0, _D, l3, (z, z, z), unroll=_UNROLL)
    acc += _relu(_sqrt(s) + _rad(5) - _rad(7) - _FM) + _reg(n1) + _reg(n2)

    def l4(d, c):
        s, na, nb = c
        dv = _dvec(d)
        a = _col(9, dv); r = _col(8, dv); b = _col(10, dv)
        t = a - r - b
        return (s + t * t, na + a * a, nb + b * b)
    s, n1, n2 = lax.fori_loop(0, _D, l4, (z, z, z), unroll=_UNROLL)
    acc += _relu(_sqrt(s) - (_rad(9) + _rad(10)) - _FM) + _reg(n1) + _reg(n2)

    def l5(d, c):
        s, na, nb = c
        dv = _dvec(d)
        a = _col(11, dv); b = _col(12, dv)
        t = b - a
        return (s + t * t, na + a * a, nb + b * b)
    s, n1, n2 = lax.fori_loop(0, _D, l5, (z, z, z), unroll=_UNROLL)
    acc += _relu((_rad(11) + _rad(12)) - _sqrt(s) + _FM) + _reg(n1) + _reg(n2)

    h_c.wait()

    def l6(d, c):
        s, n1, n2, dt = c
        dv = _dvec(d)
        a = _col(13, dv); b = _col(14, dv)
        t = b - a
        return (s + t * t, n1 + a * a, n2 + b * b, dt + a * b)
    s, n1, n2, dt = lax.fori_loop(0, _D, l6, (z, z, z, z), unroll=_UNROLL)
    direction = dt / (jnp.maximum(_sqrt(n1), jnp.float32(1e-12))
                      * jnp.maximum(_sqrt(n2), jnp.float32(1e-12)))
    acc += (_relu(_sqrt(s) - _FM) + _reg(n1) + _reg(n2)
            + jnp.abs(_F1 - direction))

    def l7(d, c):
        s, n1, n2, n3, ncd, dt = c
        dv = _dvec(d)
        a = _col(15, dv); b = _col(16, dv); e = _col(17, dv)
        t = e - a - b
        cd = a + b
        return (s + t * t, n1 + a * a, n2 + b * b, n3 + e * e,
                ncd + cd * cd, dt + cd * e)
    s, n1, n2, n3, ncd, dt = lax.fori_loop(
        0, _D, l7, (z, z, z, z, z, z), unroll=_UNROLL)
    direction = dt / (jnp.maximum(_sqrt(ncd), jnp.float32(1e-12))
                      * jnp.maximum(_sqrt(n3), jnp.float32(1e-12)))
    acc += (_relu(_sqrt(s) - _FM) + _reg(n1) + _reg(n2) + _reg(n3)
            + jnp.abs(_F1 - direction))

    def l8(d, c):
        s, na, nb = c
        dv = _dvec(d)
        a = _col(18, dv); r = _col(19, dv); b = _col(20, dv)
        t = a + r - b
        return (s + t * t, na + a * a, nb + b * b)
    s, n1, n2 = lax.fori_loop(0, _D, l8, (z, z, z), unroll=_UNROLL)
    acc += (-(_sqrt(s) - _rad(18) - _rad(20)) + _FM) + _reg(n1) + _reg(n2)

    acc = acc * jnp.float32(1.0 / _B)

    accbuf[...] = acc
    pltpu.sync_copy(accbuf, shared.at[pl.ds(sid * _L, _L)])
    plsc.subcore_barrier()

    @pl.when(sid == 0)
    def _():
        pltpu.sync_copy(shared, tmp)
        tot = tmp[pl.ds(0, _L)]
        for s_ in range(1, _NS):
            tot = tot + tmp[pl.ds(s_ * _L, _L)]
        outv[...] = jnp.broadcast_to(jnp.sum(tot), (_L,))
        pltpu.sync_copy(outv, out_h.at[cid])


@jax.jit
def _sc_loss(cidx, tab, cer):
    cp = pltpu.CompilerParams()
    if "needs_layout_passes" in pltpu.CompilerParams.__dataclass_fields__:
        cp = dataclasses.replace(cp, needs_layout_passes=False)
    run = pl.kernel(
        _body,
        out_type=jax.ShapeDtypeStruct((_NC, _L), jnp.float32),
        mesh=plsc.VectorSubcoreMesh(core_axis_name="c", subcore_axis_name="s"),
        scratch_types=[
            pltpu.VMEM((3, _GW), jnp.int32),
            pltpu.VMEM((_BPW * _NT, _D), jnp.float32),
            pltpu.VMEM((1000,), jnp.float32),
            pltpu.VMEM((_L,), jnp.float32),
            pltpu.VMEM((_NS * _L,), jnp.float32),
            pltpu.VMEM((_L,), jnp.float32),
            pltpu.VMEM_SHARED((_NS * _L,), jnp.float32),
            pltpu.SemaphoreType.DMA,
            pltpu.SemaphoreType.DMA,
            pltpu.SemaphoreType.DMA,
            pltpu.SemaphoreType.DMA,
            pltpu.SemaphoreType.DMA,
        ],
        compiler_params=cp,
    )
    return run(cidx, tab, cer)


def kernel(nf1, nf2, nf3, nf4, disjoint, role_inclusion, role_chain,
           nf3_neg, class_emb, rel_emb):
    i32 = jnp.int32
    off_crd = jnp.array([0, 1000, 0], i32)
    off_rcd = jnp.array([1000, 0, 0], i32)
    comb = jnp.concatenate([
        nf1.astype(i32),
        nf2.astype(i32),
        nf3.astype(i32) + off_crd,
        nf4.astype(i32) + off_rcd,
        disjoint.astype(i32),
        role_inclusion.astype(i32) + 1000,
        role_chain.astype(i32) + 1000,
        nf3_neg.astype(i32) + off_crd,
    ], axis=1)
    cidx = comb.reshape(_NC * _NS, _BPW, _NT).transpose(0, 2, 1)
    cidx = cidx.reshape(_NC * _NS, 3, _GW)
    tab = jnp.concatenate([class_emb[:, :_D], rel_emb], axis=0)
    out = _sc_loss(cidx, tab, class_emb[:, _D])
    return out[0, 0] + out[1, 0]


# speedup vs baseline: 1.7634x; 1.0484x over previous
import dataclasses

import jax
import jax.numpy as jnp
from jax import lax
from jax.experimental import pallas as pl
from jax.experimental.pallas import tpu as pltpu
from jax.experimental.pallas import tpu_sc as plsc

_B = 512
_D = 128
_NC = 2
_NS = 16
_L = 16
_BPW = _B // (_NC * _NS)
_NT = 21
_GW = 112
_F1 = 1.0
_FM = 0.1
_UNROLL = 8


def _rsqrt(x):
    i = lax.bitcast_convert_type(x, jnp.int32)
    y = lax.bitcast_convert_type(jnp.int32(0x5F3759DF) - (i >> 1), jnp.float32)
    for _ in range(3):
        y = y * (jnp.float32(1.5) - jnp.float32(0.5) * x * y * y)
    return y


def _sqrt(x):
    return x * _rsqrt(jnp.maximum(x, jnp.float32(1e-30)))


def _relu(x):
    return jnp.maximum(x, jnp.float32(0.0))


def _reg(n2):
    return jnp.abs(_sqrt(n2) - _F1)


def _body(cidx_h, tab_h, cer_h, out_h, blk, rows, radii, accbuf, tmp, outv,
          shared, sem_i, sem_r, sem_a, sem_b, sem_c):
    cid = lax.axis_index("c")
    sid = lax.axis_index("s")
    wid = cid * _NS + sid

    iota = lax.iota(jnp.int32, _L)

    h_idx = pltpu.async_copy(cidx_h.at[wid], blk, sem_i)
    h_rad = pltpu.async_copy(cer_h, radii, sem_r)
    h_idx.wait()
    h_a = pltpu.async_copy(tab_h.at[blk.at[0]], rows.at[pl.ds(0, _GW)], sem_a)
    h_b = pltpu.async_copy(tab_h.at[blk.at[1]], rows.at[pl.ds(_GW, _GW)],
                           sem_b)
    h_c = pltpu.async_copy(tab_h.at[blk.at[2]], rows.at[pl.ds(2 * _GW, _GW)],
                           sem_c)

    def _dvec(d):
        return (d + iota) & jnp.int32(127)

    def _col(t, dv):
        return plsc.load_gather(rows, [iota * _NT + t, dv])

    def _rad(t):
        k = iota * _NT + t
        ci = plsc.load_gather(blk, [k // _GW, k % _GW])
        return jnp.abs(plsc.load_gather(radii, [ci]))

    z = jnp.zeros((_L,), jnp.float32)

    h_a.wait()
    h_b.wait()
    h_c.wait()
    h_rad.wait()

    def l1(d, c):
        s, na, nb = c
        dv = _dvec(d)
        a = _col(0, dv); b = _col(1, dv)
        t = a - b
        return (s + t * t, na + a * a, nb + b * b)
    s, n1, n2 = lax.fori_loop(0, _D, l1, (z, z, z), unroll=_UNROLL)
    acc = _relu(_sqrt(s) + _rad(0) - _rad(1) - _FM) + _reg(n1) + _reg(n2)

    def l2(d, c):
        d21, d31, d32, n1, n2, n3 = c
        dv = _dvec(d)
        a = _col(2, dv); b = _col(3, dv); e = _col(4, dv)
        t21 = b - a; t31 = e - a; t32 = e - b
        return (d21 + t21 * t21, d31 + t31 * t31, d32 + t32 * t32,
                n1 + a * a, n2 + b * b, n3 + e * e)
    d21, d31, d32, n1, n2, n3 = lax.fori_loop(
        0, _D, l2, (z, z, z, z, z, z), unroll=_UNROLL)
    rc = _rad(2); rd = _rad(3)
    acc += (_relu(_sqrt(d21) - (rc + rd) - _FM) + _relu(_sqrt(d31) - rc - _FM)
            + _relu(_sqrt(d32) - rd - _FM) + _reg(n1) + _reg(n2) + _reg(n3))

    def l3(d, c):
        s, na, nb = c
        dv = _dvec(d)
        a = _col(5, dv); r = _col(6, dv); b = _col(7, dv)
        t = a + r - b
        return (s + t * t, na + a * a, nb + b * b)
    s, n1, n2 = lax.fori_loop(0, _D, l3, (z, z, z), unroll=_UNROLL)
    acc += _relu(_sqrt(s) + _rad(5) - _rad(7) - _FM) + _reg(n1) + _reg(n2)

    def l4(d, c):
        s, na, nb = c
        dv = _dvec(d)
        a = _col(9, dv); r = _col(8, dv); b = _col(10, dv)
        t = a - r - b
        return (s + t * t, na + a * a, nb + b * b)
    s, n1, n2 = lax.fori_loop(0, _D, l4, (z, z, z), unroll=_UNROLL)
    acc += _relu(_sqrt(s) - (_rad(9) + _rad(10)) - _FM) + _reg(n1) + _reg(n2)

    def l5(d, c):
        s, na, nb = c
        dv = _dvec(d)
        a = _col(11, dv); b = _col(12, dv)
        t = b - a
        return (s + t * t, na + a * a, nb + b * b)
    s, n1, n2 = lax.fori_loop(0, _D, l5, (z, z, z), unroll=_UNROLL)
    acc += _relu((_rad(11) + _rad(12)) - _sqrt(s) + _FM) + _reg(n1) + _reg(n2)

    def l6(d, c):
        s, n1, n2, dt = c
        dv = _dvec(d)
        a = _col(13, dv); b = _col(14, dv)
        t = b - a
        return (s + t * t, n1 + a * a, n2 + b * b, dt + a * b)
    s, n1, n2, dt = lax.fori_loop(0, _D, l6, (z, z, z, z), unroll=_UNROLL)
    direction = dt / (jnp.maximum(_sqrt(n1), jnp.float32(1e-12))
                      * jnp.maximum(_sqrt(n2), jnp.float32(1e-12)))
    acc += (_relu(_sqrt(s) - _FM) + _reg(n1) + _reg(n2)
            + jnp.abs(_F1 - direction))

    def l7(d, c):
        s, n1, n2, n3, ncd, dt = c
        dv = _dvec(d)
        a = _col(15, dv); b = _col(16, dv); e = _col(17, dv)
        t = e - a - b
        cd = a + b
        return (s + t * t, n1 + a * a, n2 + b * b, n3 + e * e,
                ncd + cd * cd, dt + cd * e)
    s, n1, n2, n3, ncd, dt = lax.fori_loop(
        0, _D, l7, (z, z, z, z, z, z), unroll=_UNROLL)
    direction = dt / (jnp.maximum(_sqrt(ncd), jnp.float32(1e-12))
                      * jnp.maximum(_sqrt(n3), jnp.float32(1e-12)))
    acc += (_relu(_sqrt(s) - _FM) + _reg(n1) + _reg(n2) + _reg(n3)
            + jnp.abs(_F1 - direction))

    def l8(d, c):
        s, na, nb = c
        dv = _dvec(d)
        a = _col(18, dv); r = _col(19, dv); b = _col(20, dv)
        t = a + r - b
        return (s + t * t, na + a * a, nb + b * b)
    s, n1, n2 = lax.fori_loop(0, _D, l8, (z, z, z), unroll=_UNROLL)
    acc += (-(_sqrt(s) - _rad(18) - _rad(20)) + _FM) + _reg(n1) + _reg(n2)

    acc = acc * jnp.float32(1.0 / _B)

    accbuf[...] = acc
    pltpu.sync_copy(accbuf, shared.at[pl.ds(sid * _L, _L)])
    plsc.subcore_barrier()

    @pl.when(sid == 0)
    def _():
        pltpu.sync_copy(shared, tmp)
        tot = tmp[pl.ds(0, _L)]
        for s_ in range(1, _NS):
            tot = tot + tmp[pl.ds(s_ * _L, _L)]
        outv[...] = jnp.broadcast_to(jnp.sum(tot), (_L,))
        pltpu.sync_copy(outv, out_h.at[cid])


@jax.jit
def _sc_loss(cidx, tab, cer):
    cp = pltpu.CompilerParams()
    if "needs_layout_passes" in pltpu.CompilerParams.__dataclass_fields__:
        cp = dataclasses.replace(cp, needs_layout_passes=False)
    run = pl.kernel(
        _body,
        out_type=jax.ShapeDtypeStruct((_NC, _L), jnp.float32),
        mesh=plsc.VectorSubcoreMesh(core_axis_name="c", subcore_axis_name="s"),
        scratch_types=[
            pltpu.VMEM((3, _GW), jnp.int32),
            pltpu.VMEM((_BPW * _NT, _D), jnp.float32),
            pltpu.VMEM((1000,), jnp.float32),
            pltpu.VMEM((_L,), jnp.float32),
            pltpu.VMEM((_NS * _L,), jnp.float32),
            pltpu.VMEM((_L,), jnp.float32),
            pltpu.VMEM_SHARED((_NS * _L,), jnp.float32),
            pltpu.SemaphoreType.DMA,
            pltpu.SemaphoreType.DMA,
            pltpu.SemaphoreType.DMA,
            pltpu.SemaphoreType.DMA,
            pltpu.SemaphoreType.DMA,
        ],
        compiler_params=cp,
    )
    return run(cidx, tab, cer)


def kernel(nf1, nf2, nf3, nf4, disjoint, role_inclusion, role_chain,
           nf3_neg, class_emb, rel_emb):
    i32 = jnp.int32
    off_crd = jnp.array([0, 1000, 0], i32)
    off_rcd = jnp.array([1000, 0, 0], i32)
    comb = jnp.concatenate([
        nf1.astype(i32),
        nf2.astype(i32),
        nf3.astype(i32) + off_crd,
        nf4.astype(i32) + off_rcd,
        disjoint.astype(i32),
        role_inclusion.astype(i32) + 1000,
        role_chain.astype(i32) + 1000,
        nf3_neg.astype(i32) + off_crd,
    ], axis=1)
    cidx = comb.reshape(_NC * _NS, 3, _GW)
    tab = jnp.concatenate([class_emb[:, :_D], rel_emb], axis=0)
    out = _sc_loss(cidx, tab, class_emb[:, _D])
    return out[0, 0] + out[1, 0]
